# baseline (speedup 1.0000x reference)
import functools

import jax
import jax.numpy as jnp
from jax import lax
from jax.experimental import pallas as pl
from jax.experimental.pallas import tpu as pltpu
from jax.experimental.pallas import tpu_sc as plsc

B = 16384
V = 49
VP = 64
DE = 3
C = 4
DO = 3
L = 16
NW = 32
BW = B // NW
TW = 512
WPW = BW // 128 * TW

_mesh = plsc.VectorSubcoreMesh(core_axis_name="c", subcore_axis_name="s")


@functools.partial(
    pl.kernel,
    out_type=(
        jax.ShapeDtypeStruct((B // 128 * TW,), jnp.float32),
        jax.ShapeDtypeStruct((B // 128 * TW,), jnp.float32),
    ),
    mesh=_mesh,
    compiler_params=pltpu.CompilerParams(
        needs_layout_passes=False, use_tc_tiling_on_sc=False
    ),
    scratch_types=[
        pltpu.VMEM((WPW,), jnp.int32),
        pltpu.VMEM((BW,), jnp.int32),
        pltpu.VMEM((208,), jnp.float32),
        pltpu.VMEM((C * DO * VP,), jnp.float32),
        pltpu.VMEM((WPW,), jnp.float32),
        pltpu.VMEM((WPW,), jnp.float32),
        pltpu.SemaphoreType.DMA,
        pltpu.SemaphoreType.DMA,
    ],
)
def _cbow_sc(xp_hbm, x1_hbm, p_hbm, yp_hbm, y1p_hbm,
             x_v, x1_v, p_v, t_v, yp_v, y1p_v, sem_in, sem_out):
    embt_v = p_v
    WOFF = 152
    nc = _mesh.num_cores
    wid = lax.axis_index("s") * nc + lax.axis_index("c")

    small_copies = [
        pltpu.async_copy(p_hbm, p_v.at[pl.ds(0, 192)], sem_in),
    ]
    big_copies = [
        pltpu.async_copy(xp_hbm.at[pl.ds(wid * WPW, WPW)], x_v, sem_in),
        pltpu.async_copy(x1_hbm.at[pl.ds(wid * BW, BW)], x1_v, sem_in),
    ]
    for cp in small_copies:
        cp.wait()

    iota = lax.iota(jnp.int32, L)
    b_vec = p_v[pl.ds(176, L)]

    def t_build(vg, carry):
        vv = vg * L + iota
        e = [plsc.load_gather(embt_v, [vv + d * V]) for d in range(DE)]

        def t_c(c, carry2):
            def t_j(j, carry3):
                widx = WOFF + j * 12 + c * DE
                zero = jnp.full((L,), 0, jnp.int32)
                acc = e[0] * plsc.load_gather(p_v, [zero + widx])
                acc = acc + e[1] * plsc.load_gather(p_v, [zero + (widx + 1)])
                acc = acc + e[2] * plsc.load_gather(p_v, [zero + (widx + 2)])
                t_v[pl.ds((c * DO + j) * VP + vg * L, L)] = acc
                return carry3

            lax.fori_loop(0, DO, t_j, 0)
            return carry2

        lax.fori_loop(0, C, t_c, 0)
        return carry

    lax.fori_loop(0, VP // L, t_build, 0)

    for cp in big_copies:
        cp.wait()

    bj = [b_vec[12 + j] for j in range(DO)]

    @plsc.parallel_loop(0, BW // L, unroll=4)
    def group(gg):
        go = (gg >> 3) * TW + (gg & 7) * L
        xoff = gg * L
        xc = [x_v[pl.ds(go + c * 128, L)] for c in range(C)]
        for j in range(DO):
            acc = plsc.load_gather(t_v, [xc[0] + (j * VP)]) + bj[j]
            for c in range(1, C):
                acc = acc + plsc.load_gather(
                    t_v, [xc[c] + ((c * DO + j) * VP)])
            yp_v[pl.ds(go + j * 128, L)] = acc
        x1c = x1_v[pl.ds(xoff, L)]
        for j in range(DE):
            y1p_v[pl.ds(go + j * 128, L)] = plsc.load_gather(
                embt_v, [x1c + j * V]
            )

    out_copies = [
        pltpu.async_copy(yp_v, yp_hbm.at[pl.ds(wid * WPW, WPW)], sem_out),
        pltpu.async_copy(y1p_v, y1p_hbm.at[pl.ds(wid * WPW, WPW)], sem_out),
    ]
    for cp in out_copies:
        cp.wait()


def _unphys(flat):
    return flat.reshape(B // 128, 4, 128).transpose(0, 2, 1).reshape(B, 4)[:, :DO]


def kernel(x, x1, emb, W, b):
    xp = x.astype(jnp.int32).reshape(B // 128, 128, 4).transpose(0, 2, 1).reshape(-1)
    packed = jnp.concatenate(
        [emb.T.reshape(-1), jnp.zeros((5,), jnp.float32), W.reshape(-1), b,
         jnp.zeros((1,), jnp.float32)]
    )
    yp, y1p = _cbow_sc(xp, x1.astype(jnp.int32), packed)
    return (_unphys(yp), _unphys(y1p))


# speedup vs baseline: 1.0185x; 1.0185x over previous
import functools

import jax
import jax.numpy as jnp
from jax import lax
from jax.experimental import pallas as pl
from jax.experimental.pallas import tpu as pltpu
from jax.experimental.pallas import tpu_sc as plsc

B = 16384
V = 49
VP = 64
DE = 3
C = 4
DO = 3
L = 16
NW = 32
BW = B // NW
TW = 512
WPW = BW // 128 * TW

_mesh = plsc.VectorSubcoreMesh(core_axis_name="c", subcore_axis_name="s")


@functools.partial(
    pl.kernel,
    out_type=(
        jax.ShapeDtypeStruct((B // 128 * TW,), jnp.float32),
        jax.ShapeDtypeStruct((B // 128 * TW,), jnp.float32),
    ),
    mesh=_mesh,
    compiler_params=pltpu.CompilerParams(
        needs_layout_passes=False, use_tc_tiling_on_sc=False
    ),
    scratch_types=[
        pltpu.VMEM((WPW,), jnp.int32),
        pltpu.VMEM((BW,), jnp.int32),
        pltpu.VMEM((208,), jnp.float32),
        pltpu.VMEM((C * DO * VP,), jnp.float32),
        pltpu.VMEM((WPW,), jnp.float32),
        pltpu.VMEM((WPW,), jnp.float32),
        pltpu.SemaphoreType.DMA,
        pltpu.SemaphoreType.DMA,
    ],
)
def _cbow_sc(xp_hbm, x1_hbm, p_hbm, yp_hbm, y1p_hbm,
             x_v, x1_v, p_v, t_v, yp_v, y1p_v, sem_in, sem_out):
    embt_v = p_v
    WOFF = 152
    nc = _mesh.num_cores
    wid = lax.axis_index("s") * nc + lax.axis_index("c")

    small_copies = [
        pltpu.async_copy(p_hbm, p_v.at[pl.ds(0, 192)], sem_in),
    ]
    big_copies = [
        pltpu.async_copy(xp_hbm.at[pl.ds(wid * WPW, WPW)], x_v, sem_in),
        pltpu.async_copy(x1_hbm.at[pl.ds(wid * BW, BW)], x1_v, sem_in),
    ]
    for cp in small_copies:
        cp.wait()

    iota = lax.iota(jnp.int32, L)
    b_vec = p_v[pl.ds(176, L)]

    @plsc.parallel_loop(0, VP // L)
    def t_build(vg):
        vv = vg * L + iota
        e = [plsc.load_gather(embt_v, [vv + d * V]) for d in range(DE)]

        @plsc.parallel_loop(0, C * DO)
        def t_cj(cj):
            widx = WOFF + (cj % DO) * 12 + (cj // DO) * DE
            zero = jnp.full((L,), 0, jnp.int32)
            acc = e[0] * plsc.load_gather(p_v, [zero + widx])
            acc = acc + e[1] * plsc.load_gather(p_v, [zero + (widx + 1)])
            acc = acc + e[2] * plsc.load_gather(p_v, [zero + (widx + 2)])
            t_v[pl.ds(cj * VP + vg * L, L)] = acc

    for cp in big_copies:
        cp.wait()

    bj = [b_vec[12 + j] for j in range(DO)]

    @plsc.parallel_loop(0, BW // L, unroll=2)
    def group(gg):
        go = (gg >> 3) * TW + (gg & 7) * L
        xoff = gg * L
        xc = [x_v[pl.ds(go + c * 128, L)] for c in range(C)]
        for j in range(DO):
            acc = plsc.load_gather(t_v, [xc[0] + (j * VP)]) + bj[j]
            for c in range(1, C):
                acc = acc + plsc.load_gather(
                    t_v, [xc[c] + ((c * DO + j) * VP)])
            yp_v[pl.ds(go + j * 128, L)] = acc
        x1c = x1_v[pl.ds(xoff, L)]
        for j in range(DE):
            y1p_v[pl.ds(go + j * 128, L)] = plsc.load_gather(
                embt_v, [x1c + j * V]
            )

    out_copies = [
        pltpu.async_copy(yp_v, yp_hbm.at[pl.ds(wid * WPW, WPW)], sem_out),
        pltpu.async_copy(y1p_v, y1p_hbm.at[pl.ds(wid * WPW, WPW)], sem_out),
    ]
    for cp in out_copies:
        cp.wait()


def _unphys(flat):
    return flat.reshape(B // 128, 4, 128).transpose(0, 2, 1).reshape(B, 4)[:, :DO]


def kernel(x, x1, emb, W, b):
    xp = x.astype(jnp.int32).reshape(B // 128, 128, 4).transpose(0, 2, 1).reshape(-1)
    packed = jnp.concatenate(
        [emb.T.reshape(-1), jnp.zeros((5,), jnp.float32), W.reshape(-1), b,
         jnp.zeros((1,), jnp.float32)]
    )
    yp, y1p = _cbow_sc(xp, x1.astype(jnp.int32), packed)
    return (_unphys(yp), _unphys(y1p))
